# Initial kernel scaffold; baseline (speedup 1.0000x reference)
import jax
import jax.numpy as jnp
from jax.experimental import pallas as pl


def kernel(x, edge_index, W1, b1, W2, b2):
    raise NotImplementedError("write your pallas kernel here")


# speedup vs baseline: 44.2351x; 44.2351x over previous
import functools

import jax
import jax.numpy as jnp
from jax import lax
from jax.experimental import pallas as pl
from jax.experimental.pallas import tpu as pltpu
from jax.experimental.pallas import tpu_sc as plsc

N_NODES = 100000
N_EDGES = 6400000
IN_CH = 4
HID_CH = 32

NC = 2
NS = 16
NW = NC * NS

N_PAD = 100352
RPT = N_PAD // NS

C = 1024
STEPS = 196
EPW = STEPS * C
E_PAD = EPW * NW

_mesh = plsc.VectorSubcoreMesh(core_axis_name="c", subcore_axis_name="s")
_sc_params = pltpu.CompilerParams(use_tc_tiling_on_sc=False)


@functools.partial(
    pl.kernel,
    out_type=jax.ShapeDtypeStruct((NC, N_PAD), jnp.float32),
    mesh=_mesh,
    compiler_params=_sc_params,
    scratch_types=[
        pltpu.VMEM_SHARED((N_PAD,), jnp.float32),
        pltpu.VMEM((C,), jnp.int32),
        pltpu.VMEM((C,), jnp.float32),
        pltpu.VMEM((RPT,), jnp.float32),
    ],
)
def _sc_deg(dst_hbm, zeros_hbm, out_hbm, acc_sh, idx_v, ones_v, stage_v):
    c = lax.axis_index("c")
    s = lax.axis_index("s")
    wid = s * NC + c
    sl = pl.ds(s * RPT, RPT)

    def fill_ones(i, _):
        ones_v[pl.ds(i * 16, 16)] = jnp.ones((16,), jnp.float32)
        return 0
    lax.fori_loop(0, C // 16, fill_ones, 0, unroll=4)

    pltpu.sync_copy(zeros_hbm.at[sl], stage_v)
    pltpu.sync_copy(stage_v, acc_sh.at[sl])
    plsc.subcore_barrier()

    base = wid * EPW

    def step(t, _):
        off = base + t * C
        pltpu.sync_copy(dst_hbm.at[pl.ds(off, C)], idx_v)
        pltpu.sync_copy(ones_v, acc_sh.at[idx_v], add=True)
        return 0

    lax.fori_loop(0, STEPS, step, 0)
    plsc.subcore_barrier()

    pltpu.sync_copy(acc_sh.at[sl], stage_v)
    pltpu.sync_copy(stage_v, out_hbm.at[c, sl])


def _make_agg(nch):
    n_in = 2 + nch + 1

    @functools.partial(
        pl.kernel,
        out_type=jax.ShapeDtypeStruct((NC, nch, N_PAD), jnp.float32),
        mesh=_mesh,
        compiler_params=_sc_params,
        scratch_types=[
            pltpu.VMEM_SHARED((N_PAD,), jnp.float32),
            pltpu.VMEM((C,), jnp.int32),
            pltpu.VMEM((C,), jnp.int32),
            pltpu.VMEM((C,), jnp.float32),
            pltpu.VMEM((RPT,), jnp.float32),
            pltpu.SemaphoreType.DMA,
        ],
    )
    def agg(*refs):
        src_hbm, dst_hbm = refs[0], refs[1]
        tabs = refs[2:2 + nch]
        zeros_hbm = refs[2 + nch]
        out_hbm = refs[n_in]
        acc_sh, sidx_v, didx_v, rows_v, stage_v, sem = refs[n_in + 1:]

        c = lax.axis_index("c")
        s = lax.axis_index("s")
        wid = s * NC + c
        sl = pl.ds(s * RPT, RPT)
        base = wid * EPW

        for k in range(nch):
            pltpu.sync_copy(zeros_hbm.at[sl], stage_v)
            pltpu.sync_copy(stage_v, acc_sh.at[sl])
            plsc.subcore_barrier()

            tab_hbm = tabs[k]

            def step(t, _):
                off = base + t * C
                pltpu.sync_copy(src_hbm.at[pl.ds(off, C)], sidx_v)
                pltpu.sync_copy(dst_hbm.at[pl.ds(off, C)], didx_v)
                pltpu.async_copy(tab_hbm.at[sidx_v], rows_v, sem).wait()
                pltpu.sync_copy(rows_v, acc_sh.at[didx_v], add=True)
                return 0

            lax.fori_loop(0, STEPS, step, 0)
            plsc.subcore_barrier()

            pltpu.sync_copy(acc_sh.at[sl], stage_v)
            pltpu.sync_copy(stage_v, out_hbm.at[c, k, sl])
            plsc.subcore_barrier()

    return agg


_sc_agg4 = _make_agg(IN_CH)
_sc_agg1 = _make_agg(1)


_BR = 6272
_GRID = N_PAD // _BR


def _tc_prep_body(deg0, deg1, x, t1, dinv):
    di = 1.0 / jnp.sqrt(deg0[...] + deg1[...] + 1.0)
    dinv[...] = di
    t1[...] = x[...] * di


_tc_prep = pl.pallas_call(
    _tc_prep_body,
    grid=(_GRID,),
    in_specs=[
        pl.BlockSpec((_BR, 1), lambda i: (i, 0)),
        pl.BlockSpec((_BR, 1), lambda i: (i, 0)),
        pl.BlockSpec((_BR, IN_CH), lambda i: (i, 0)),
    ],
    out_specs=[
        pl.BlockSpec((_BR, IN_CH), lambda i: (i, 0)),
        pl.BlockSpec((_BR, 1), lambda i: (i, 0)),
    ],
    out_shape=[
        jax.ShapeDtypeStruct((N_PAD, IN_CH), jnp.float32),
        jax.ShapeDtypeStruct((N_PAD, 1), jnp.float32),
    ],
)


def _tc_mid_body(z0, z1, t1, dinv, W1, b1, W2, t2):
    z = (z0[...] + z1[...] + t1[...]) * dinv[...]
    h = jnp.maximum(
        jnp.dot(z, W1[...], preferred_element_type=jnp.float32) + b1[...], 0.0)
    u = jnp.dot(h, W2[...], preferred_element_type=jnp.float32)
    t2[...] = u * dinv[...]


_tc_mid = pl.pallas_call(
    _tc_mid_body,
    grid=(_GRID,),
    in_specs=[
        pl.BlockSpec((_BR, IN_CH), lambda i: (i, 0)),
        pl.BlockSpec((_BR, IN_CH), lambda i: (i, 0)),
        pl.BlockSpec((_BR, IN_CH), lambda i: (i, 0)),
        pl.BlockSpec((_BR, 1), lambda i: (i, 0)),
        pl.BlockSpec((IN_CH, HID_CH), lambda i: (0, 0)),
        pl.BlockSpec((1, HID_CH), lambda i: (0, 0)),
        pl.BlockSpec((HID_CH, 1), lambda i: (0, 0)),
    ],
    out_specs=pl.BlockSpec((_BR, 1), lambda i: (i, 0)),
    out_shape=jax.ShapeDtypeStruct((N_PAD, 1), jnp.float32),
)


def _tc_out_body(v0, v1, t2, dinv, b2, o):
    val = (v0[...] + v1[...] + t2[...]) * dinv[...] + b2[0, 0]
    o[...] = 1.0 / (1.0 + jnp.exp(-val))


_tc_out = pl.pallas_call(
    _tc_out_body,
    grid=(_GRID,),
    in_specs=[
        pl.BlockSpec((_BR, 1), lambda i: (i, 0)),
        pl.BlockSpec((_BR, 1), lambda i: (i, 0)),
        pl.BlockSpec((_BR, 1), lambda i: (i, 0)),
        pl.BlockSpec((_BR, 1), lambda i: (i, 0)),
        pl.BlockSpec((1, 1), lambda i: (0, 0)),
    ],
    out_specs=pl.BlockSpec((_BR, 1), lambda i: (i, 0)),
    out_shape=jax.ShapeDtypeStruct((N_PAD, 1), jnp.float32),
)


def kernel(x, edge_index, W1, b1, W2, b2):
    ei = edge_index.astype(jnp.int32)
    n_extra = E_PAD - N_EDGES
    pad_idx = N_NODES + (lax.iota(jnp.int32, n_extra) % 128)
    srcf = jnp.concatenate([ei[0], pad_idx])
    dstf = jnp.concatenate([ei[1], pad_idx])
    xp = jnp.pad(x, ((0, N_PAD - N_NODES), (0, 0)))
    zeros_n = jnp.zeros((N_PAD,), jnp.float32)

    deg_part = _sc_deg(dstf, zeros_n)
    t1, dinv = _tc_prep(deg_part[0][:, None], deg_part[1][:, None], xp)

    t1T = t1.T
    z4 = _sc_agg4(srcf, dstf, t1T[0], t1T[1], t1T[2], t1T[3], zeros_n)
    t2 = _tc_mid(z4[0].T, z4[1].T, t1, dinv,
                 W1, b1.reshape(1, HID_CH), W2)

    v = _sc_agg1(srcf, dstf, t2.reshape(-1), zeros_n)
    out = _tc_out(v[0, 0][:, None], v[1, 0][:, None], t2, dinv,
                  b2.reshape(1, 1))
    return out[:N_NODES, 0]


# speedup vs baseline: 113.4160x; 2.5639x over previous
import functools

import jax
import jax.numpy as jnp
from jax import lax
from jax.experimental import pallas as pl
from jax.experimental.pallas import tpu as pltpu
from jax.experimental.pallas import tpu_sc as plsc

N_NODES = 100000
N_EDGES = 6400000
IN_CH = 4
HID_CH = 32

NC = 2
NS = 16
NW = NC * NS

N_PAD = 100352
RPT = N_PAD // NS

C = 1024
STEPS = 196
EPW = STEPS * C
E_PAD = EPW * NW

_mesh = plsc.VectorSubcoreMesh(core_axis_name="c", subcore_axis_name="s")
_sc_params = pltpu.CompilerParams(use_tc_tiling_on_sc=False)


@functools.partial(
    pl.kernel,
    out_type=jax.ShapeDtypeStruct((NC, N_PAD), jnp.float32),
    mesh=_mesh,
    compiler_params=_sc_params,
    scratch_types=[
        pltpu.VMEM_SHARED((N_PAD,), jnp.float32),
        pltpu.VMEM((C,), jnp.int32),
        pltpu.VMEM((C,), jnp.int32),
        pltpu.VMEM((C,), jnp.float32),
        pltpu.VMEM((RPT,), jnp.float32),
        pltpu.SemaphoreType.DMA,
        pltpu.SemaphoreType.DMA,
    ],
)
def _sc_deg(dst_hbm, zeros_hbm, out_hbm, acc_sh, idx0_v, idx1_v, ones_v,
            stage_v, sem0, sem1):
    c = lax.axis_index("c")
    s = lax.axis_index("s")
    wid = s * NC + c
    sl = pl.ds(s * RPT, RPT)
    idx_v = (idx0_v, idx1_v)
    sems = (sem0, sem1)

    def fill_ones(i, _):
        ones_v[pl.ds(i * 16, 16)] = jnp.ones((16,), jnp.float32)
        return 0
    lax.fori_loop(0, C // 16, fill_ones, 0, unroll=4)

    pltpu.sync_copy(zeros_hbm.at[sl], stage_v)
    pltpu.sync_copy(stage_v, acc_sh.at[sl])
    plsc.subcore_barrier()

    base = wid * EPW
    for b in range(2):
        pltpu.async_copy(dst_hbm.at[pl.ds(base + b * C, C)], idx_v[b], sems[b])

    def rnd(r, _):
        for b in range(2):
            pltpu.make_async_copy(
                dst_hbm.at[pl.ds(0, C)], idx_v[b], sems[b]).wait()
            pltpu.sync_copy(ones_v, acc_sh.at[idx_v[b]], add=True)
        t_next = jnp.minimum((r + 1) * 2, STEPS - 2)
        for b in range(2):
            pltpu.async_copy(
                dst_hbm.at[pl.ds(base + (t_next + b) * C, C)], idx_v[b], sems[b])
        return 0

    lax.fori_loop(0, STEPS // 2, rnd, 0)
    for b in range(2):
        pltpu.make_async_copy(dst_hbm.at[pl.ds(0, C)], idx_v[b], sems[b]).wait()
    plsc.subcore_barrier()

    pltpu.sync_copy(acc_sh.at[sl], stage_v)
    pltpu.sync_copy(stage_v, out_hbm.at[c, sl])


_NB = 4


def _make_agg(nch):
    n_in = 2 + nch + 1
    scratch = (
        [pltpu.VMEM_SHARED((N_PAD,), jnp.float32)] * 2
        + [pltpu.VMEM((C,), jnp.int32)] * (2 * _NB)
        + [pltpu.VMEM((C,), jnp.float32)] * _NB
        + [pltpu.VMEM((RPT,), jnp.float32)]
        + [pltpu.SemaphoreType.DMA] * (2 * _NB)
    )

    @functools.partial(
        pl.kernel,
        out_type=jax.ShapeDtypeStruct((NC, nch, N_PAD), jnp.float32),
        mesh=_mesh,
        compiler_params=_sc_params,
        scratch_types=scratch,
    )
    def agg(*refs):
        src_hbm, dst_hbm = refs[0], refs[1]
        tabs = refs[2:2 + nch]
        zeros_hbm = refs[2 + nch]
        out_hbm = refs[n_in]
        sc = list(refs[n_in + 1:])
        acc_sh, tab_sh = sc[0], sc[1]
        sidx = sc[2:2 + _NB]
        didx = sc[2 + _NB:2 + 2 * _NB]
        rows = sc[2 + 2 * _NB:2 + 3 * _NB]
        stage_v = sc[2 + 3 * _NB]
        sem_i = sc[3 + 3 * _NB:3 + 4 * _NB]
        sem_g = sc[3 + 4 * _NB:3 + 5 * _NB]

        c = lax.axis_index("c")
        s = lax.axis_index("s")
        wid = s * NC + c
        sl = pl.ds(s * RPT, RPT)
        base = wid * EPW

        def fire_idx(b, t):
            off = base + t * C
            pltpu.async_copy(src_hbm.at[pl.ds(off, C)], sidx[b], sem_i[b])
            pltpu.async_copy(dst_hbm.at[pl.ds(off, C)], didx[b], sem_i[b])

        def wait_idx(b):
            pltpu.make_async_copy(
                src_hbm.at[pl.ds(0, C)], sidx[b], sem_i[b]).wait()
            pltpu.make_async_copy(
                dst_hbm.at[pl.ds(0, C)], didx[b], sem_i[b]).wait()

        for k in range(nch):
            pltpu.sync_copy(zeros_hbm.at[sl], stage_v)
            pltpu.sync_copy(stage_v, acc_sh.at[sl])
            pltpu.sync_copy(tabs[k].at[sl], stage_v)
            pltpu.sync_copy(stage_v, tab_sh.at[sl])
            plsc.subcore_barrier()

            for b in range(_NB):
                fire_idx(b, b)

            def rnd(r, _):
                descs = []
                for b in range(_NB):
                    wait_idx(b)
                    descs.append(
                        pltpu.async_copy(tab_sh.at[sidx[b]], rows[b], sem_g[b]))
                for b in range(_NB):
                    descs[b].wait()
                    pltpu.sync_copy(rows[b], acc_sh.at[didx[b]], add=True)
                t_next = jnp.minimum((r + 1) * _NB, STEPS - _NB)
                for b in range(_NB):
                    fire_idx(b, t_next + b)
                return 0

            lax.fori_loop(0, STEPS // _NB, rnd, 0)
            for b in range(_NB):
                wait_idx(b)
            plsc.subcore_barrier()

            pltpu.sync_copy(acc_sh.at[sl], stage_v)
            pltpu.sync_copy(stage_v, out_hbm.at[c, k, sl])
            plsc.subcore_barrier()

    return agg


_sc_agg4 = _make_agg(IN_CH)
_sc_agg1 = _make_agg(1)


_BR = 6272
_GRID = N_PAD // _BR


def _tc_prep_body(deg0, deg1, x, t1, dinv):
    di = 1.0 / jnp.sqrt(deg0[...] + deg1[...] + 1.0)
    dinv[...] = di
    t1[...] = x[...] * di


_tc_prep = pl.pallas_call(
    _tc_prep_body,
    grid=(_GRID,),
    in_specs=[
        pl.BlockSpec((_BR, 1), lambda i: (i, 0)),
        pl.BlockSpec((_BR, 1), lambda i: (i, 0)),
        pl.BlockSpec((_BR, IN_CH), lambda i: (i, 0)),
    ],
    out_specs=[
        pl.BlockSpec((_BR, IN_CH), lambda i: (i, 0)),
        pl.BlockSpec((_BR, 1), lambda i: (i, 0)),
    ],
    out_shape=[
        jax.ShapeDtypeStruct((N_PAD, IN_CH), jnp.float32),
        jax.ShapeDtypeStruct((N_PAD, 1), jnp.float32),
    ],
)


def _tc_mid_body(z0, z1, t1, dinv, W1, b1, W2, t2):
    z = (z0[...] + z1[...] + t1[...]) * dinv[...]
    h = jnp.maximum(
        jnp.dot(z, W1[...], preferred_element_type=jnp.float32) + b1[...], 0.0)
    u = jnp.dot(h, W2[...], preferred_element_type=jnp.float32)
    t2[...] = u * dinv[...]


_tc_mid = pl.pallas_call(
    _tc_mid_body,
    grid=(_GRID,),
    in_specs=[
        pl.BlockSpec((_BR, IN_CH), lambda i: (i, 0)),
        pl.BlockSpec((_BR, IN_CH), lambda i: (i, 0)),
        pl.BlockSpec((_BR, IN_CH), lambda i: (i, 0)),
        pl.BlockSpec((_BR, 1), lambda i: (i, 0)),
        pl.BlockSpec((IN_CH, HID_CH), lambda i: (0, 0)),
        pl.BlockSpec((1, HID_CH), lambda i: (0, 0)),
        pl.BlockSpec((HID_CH, 1), lambda i: (0, 0)),
    ],
    out_specs=pl.BlockSpec((_BR, 1), lambda i: (i, 0)),
    out_shape=jax.ShapeDtypeStruct((N_PAD, 1), jnp.float32),
)


def _tc_out_body(v0, v1, t2, dinv, b2, o):
    val = (v0[...] + v1[...] + t2[...]) * dinv[...] + b2[0, 0]
    o[...] = 1.0 / (1.0 + jnp.exp(-val))


_tc_out = pl.pallas_call(
    _tc_out_body,
    grid=(_GRID,),
    in_specs=[
        pl.BlockSpec((_BR, 1), lambda i: (i, 0)),
        pl.BlockSpec((_BR, 1), lambda i: (i, 0)),
        pl.BlockSpec((_BR, 1), lambda i: (i, 0)),
        pl.BlockSpec((_BR, 1), lambda i: (i, 0)),
        pl.BlockSpec((1, 1), lambda i: (0, 0)),
    ],
    out_specs=pl.BlockSpec((_BR, 1), lambda i: (i, 0)),
    out_shape=jax.ShapeDtypeStruct((N_PAD, 1), jnp.float32),
)


def kernel(x, edge_index, W1, b1, W2, b2):
    ei = edge_index.astype(jnp.int32)
    n_extra = E_PAD - N_EDGES
    pad_idx = N_NODES + (lax.iota(jnp.int32, n_extra) % 128)
    srcf = jnp.concatenate([ei[0], pad_idx])
    dstf = jnp.concatenate([ei[1], pad_idx])
    xp = jnp.pad(x, ((0, N_PAD - N_NODES), (0, 0)))
    zeros_n = jnp.zeros((N_PAD,), jnp.float32)

    deg_part = _sc_deg(dstf, zeros_n)
    t1, dinv = _tc_prep(deg_part[0][:, None], deg_part[1][:, None], xp)

    t1T = t1.T
    z4 = _sc_agg4(srcf, dstf, t1T[0], t1T[1], t1T[2], t1T[3], zeros_n)
    t2 = _tc_mid(z4[0].T, z4[1].T, t1, dinv,
                 W1, b1.reshape(1, HID_CH), W2)

    v = _sc_agg1(srcf, dstf, t2.reshape(-1), zeros_n)
    out = _tc_out(v[0, 0][:, None], v[1, 0][:, None], t2, dinv,
                  b2.reshape(1, 1))
    return out[:N_NODES, 0]


# speedup vs baseline: 126.6518x; 1.1167x over previous
import functools

import jax
import jax.numpy as jnp
from jax import lax
from jax.experimental import pallas as pl
from jax.experimental.pallas import tpu as pltpu
from jax.experimental.pallas import tpu_sc as plsc

N_NODES = 100000
N_EDGES = 6400000
IN_CH = 4
HID_CH = 32

NC = 2
NS = 16
NW = NC * NS

N_PAD = 100352
RPT = N_PAD // NS

C = 2000
STEPS = 100
EPW = STEPS * C

_mesh = plsc.VectorSubcoreMesh(core_axis_name="c", subcore_axis_name="s")
_sc_params = pltpu.CompilerParams(use_tc_tiling_on_sc=False)


@functools.partial(
    pl.kernel,
    out_type=jax.ShapeDtypeStruct((NC, N_PAD), jnp.float32),
    mesh=_mesh,
    compiler_params=_sc_params,
    scratch_types=[
        pltpu.VMEM_SHARED((N_PAD,), jnp.float32),
        pltpu.VMEM((C,), jnp.int32),
        pltpu.VMEM((C,), jnp.int32),
        pltpu.VMEM((C,), jnp.float32),
        pltpu.VMEM((RPT,), jnp.float32),
        pltpu.SemaphoreType.DMA,
        pltpu.SemaphoreType.DMA,
    ],
)
def _sc_deg(dst_hbm, zeros_hbm, out_hbm, acc_sh, idx0_v, idx1_v, ones_v,
            stage_v, sem0, sem1):
    c = lax.axis_index("c")
    s = lax.axis_index("s")
    wid = s * NC + c
    sl = pl.ds(s * RPT, RPT)
    idx_v = (idx0_v, idx1_v)
    sems = (sem0, sem1)

    def fill_ones(i, _):
        ones_v[pl.ds(i * 16, 16)] = jnp.ones((16,), jnp.float32)
        return 0
    lax.fori_loop(0, C // 16, fill_ones, 0, unroll=4)

    pltpu.sync_copy(zeros_hbm.at[sl], stage_v)
    pltpu.sync_copy(stage_v, acc_sh.at[sl])
    plsc.subcore_barrier()

    base = wid * EPW
    for b in range(2):
        pltpu.async_copy(dst_hbm.at[pl.ds(base + b * C, C)], idx_v[b], sems[b])

    def rnd(r, _):
        for b in range(2):
            pltpu.make_async_copy(
                dst_hbm.at[pl.ds(0, C)], idx_v[b], sems[b]).wait()
            pltpu.sync_copy(ones_v, acc_sh.at[idx_v[b]], add=True)
        t_next = jnp.minimum((r + 1) * 2, STEPS - 2)
        for b in range(2):
            pltpu.async_copy(
                dst_hbm.at[pl.ds(base + (t_next + b) * C, C)], idx_v[b], sems[b])
        return 0

    lax.fori_loop(0, STEPS // 2, rnd, 0)
    for b in range(2):
        pltpu.make_async_copy(dst_hbm.at[pl.ds(0, C)], idx_v[b], sems[b]).wait()
    plsc.subcore_barrier()

    pltpu.sync_copy(acc_sh.at[sl], stage_v)
    pltpu.sync_copy(stage_v, out_hbm.at[c, sl])


_NB = 4


def _make_agg(nch):
    n_in = 2 + nch + 1
    scratch = (
        [pltpu.VMEM_SHARED((N_PAD,), jnp.float32)] * 2
        + [pltpu.VMEM((C,), jnp.int32)] * (2 * _NB)
        + [pltpu.VMEM((C,), jnp.float32)] * _NB
        + [pltpu.VMEM((RPT,), jnp.float32)]
        + [pltpu.SemaphoreType.DMA] * (2 * _NB)
    )

    @functools.partial(
        pl.kernel,
        out_type=jax.ShapeDtypeStruct((NC, nch, N_PAD), jnp.float32),
        mesh=_mesh,
        compiler_params=_sc_params,
        scratch_types=scratch,
    )
    def agg(*refs):
        src_hbm, dst_hbm = refs[0], refs[1]
        tabs = refs[2:2 + nch]
        zeros_hbm = refs[2 + nch]
        out_hbm = refs[n_in]
        sc = list(refs[n_in + 1:])
        acc_sh, tab_sh = sc[0], sc[1]
        sidx = sc[2:2 + _NB]
        didx = sc[2 + _NB:2 + 2 * _NB]
        rows = sc[2 + 2 * _NB:2 + 3 * _NB]
        stage_v = sc[2 + 3 * _NB]
        sem_i = sc[3 + 3 * _NB:3 + 4 * _NB]
        sem_g = sc[3 + 4 * _NB:3 + 5 * _NB]

        c = lax.axis_index("c")
        s = lax.axis_index("s")
        wid = s * NC + c
        sl = pl.ds(s * RPT, RPT)
        base = wid * EPW

        def fire_idx(b, t):
            off = base + t * C
            pltpu.async_copy(src_hbm.at[pl.ds(off, C)], sidx[b], sem_i[b])
            pltpu.async_copy(dst_hbm.at[pl.ds(off, C)], didx[b], sem_i[b])

        def wait_idx(b):
            pltpu.make_async_copy(
                src_hbm.at[pl.ds(0, C)], sidx[b], sem_i[b]).wait()
            pltpu.make_async_copy(
                dst_hbm.at[pl.ds(0, C)], didx[b], sem_i[b]).wait()

        for k in range(nch):
            pltpu.sync_copy(zeros_hbm.at[sl], stage_v)
            pltpu.sync_copy(stage_v, acc_sh.at[sl])
            pltpu.sync_copy(tabs[k].at[sl], stage_v)
            pltpu.sync_copy(stage_v, tab_sh.at[sl])
            plsc.subcore_barrier()

            for b in range(_NB):
                fire_idx(b, b)

            def rnd(r, _):
                descs = []
                for b in range(_NB):
                    wait_idx(b)
                    descs.append(
                        pltpu.async_copy(tab_sh.at[sidx[b]], rows[b], sem_g[b]))
                for b in range(_NB):
                    descs[b].wait()
                    pltpu.sync_copy(rows[b], acc_sh.at[didx[b]], add=True)
                t_next = jnp.minimum((r + 1) * _NB, STEPS - _NB)
                for b in range(_NB):
                    fire_idx(b, t_next + b)
                return 0

            lax.fori_loop(0, STEPS // _NB, rnd, 0)
            for b in range(_NB):
                wait_idx(b)
            plsc.subcore_barrier()

            pltpu.sync_copy(acc_sh.at[sl], stage_v)
            pltpu.sync_copy(stage_v, out_hbm.at[c, k, sl])
            plsc.subcore_barrier()

    return agg


_sc_agg4 = _make_agg(IN_CH)
_sc_agg1 = _make_agg(1)


_BR = 6272
_GRID = N_PAD // _BR


def _tc_prep_body(deg0, deg1, x, t1, dinv):
    di = 1.0 / jnp.sqrt(deg0[...] + deg1[...] + 1.0)
    dinv[...] = di
    t1[...] = x[...] * di


_tc_prep = pl.pallas_call(
    _tc_prep_body,
    grid=(_GRID,),
    in_specs=[
        pl.BlockSpec((_BR, 1), lambda i: (i, 0)),
        pl.BlockSpec((_BR, 1), lambda i: (i, 0)),
        pl.BlockSpec((_BR, IN_CH), lambda i: (i, 0)),
    ],
    out_specs=[
        pl.BlockSpec((_BR, IN_CH), lambda i: (i, 0)),
        pl.BlockSpec((_BR, 1), lambda i: (i, 0)),
    ],
    out_shape=[
        jax.ShapeDtypeStruct((N_PAD, IN_CH), jnp.float32),
        jax.ShapeDtypeStruct((N_PAD, 1), jnp.float32),
    ],
)


def _tc_mid_body(z0, z1, t1, dinv, W1, b1, W2, t2):
    z = (z0[...] + z1[...] + t1[...]) * dinv[...]
    h = jnp.maximum(
        jnp.dot(z, W1[...], preferred_element_type=jnp.float32) + b1[...], 0.0)
    u = jnp.dot(h, W2[...], preferred_element_type=jnp.float32)
    t2[...] = u * dinv[...]


_tc_mid = pl.pallas_call(
    _tc_mid_body,
    grid=(_GRID,),
    in_specs=[
        pl.BlockSpec((_BR, IN_CH), lambda i: (i, 0)),
        pl.BlockSpec((_BR, IN_CH), lambda i: (i, 0)),
        pl.BlockSpec((_BR, IN_CH), lambda i: (i, 0)),
        pl.BlockSpec((_BR, 1), lambda i: (i, 0)),
        pl.BlockSpec((IN_CH, HID_CH), lambda i: (0, 0)),
        pl.BlockSpec((1, HID_CH), lambda i: (0, 0)),
        pl.BlockSpec((HID_CH, 1), lambda i: (0, 0)),
    ],
    out_specs=pl.BlockSpec((_BR, 1), lambda i: (i, 0)),
    out_shape=jax.ShapeDtypeStruct((N_PAD, 1), jnp.float32),
)


def _tc_out_body(v0, v1, t2, dinv, b2, o):
    val = (v0[...] + v1[...] + t2[...]) * dinv[...] + b2[0, 0]
    o[...] = 1.0 / (1.0 + jnp.exp(-val))


_tc_out = pl.pallas_call(
    _tc_out_body,
    grid=(_GRID,),
    in_specs=[
        pl.BlockSpec((_BR, 1), lambda i: (i, 0)),
        pl.BlockSpec((_BR, 1), lambda i: (i, 0)),
        pl.BlockSpec((_BR, 1), lambda i: (i, 0)),
        pl.BlockSpec((_BR, 1), lambda i: (i, 0)),
        pl.BlockSpec((1, 1), lambda i: (0, 0)),
    ],
    out_specs=pl.BlockSpec((_BR, 1), lambda i: (i, 0)),
    out_shape=jax.ShapeDtypeStruct((N_PAD, 1), jnp.float32),
)


def kernel(x, edge_index, W1, b1, W2, b2):
    ei = edge_index.astype(jnp.int32)
    srcf = ei[0]
    dstf = ei[1]
    xp = jnp.pad(x, ((0, N_PAD - N_NODES), (0, 0)))
    zeros_n = jnp.zeros((N_PAD,), jnp.float32)

    deg_part = _sc_deg(dstf, zeros_n)
    t1, dinv = _tc_prep(deg_part[0][:, None], deg_part[1][:, None], xp)

    t1T = t1.T
    z4 = _sc_agg4(srcf, dstf, t1T[0], t1T[1], t1T[2], t1T[3], zeros_n)
    t2 = _tc_mid(z4[0].T, z4[1].T, t1, dinv,
                 W1, b1.reshape(1, HID_CH), W2)

    v = _sc_agg1(srcf, dstf, t2.reshape(-1), zeros_n)
    out = _tc_out(v[0, 0][:, None], v[1, 0][:, None], t2, dinv,
                  b2.reshape(1, 1))
    return out[:N_NODES, 0]
